# Initial kernel scaffold; baseline (speedup 1.0000x reference)
import jax
import jax.numpy as jnp
from jax.experimental import pallas as pl


def kernel(x, edge_index, batch, W1, b1, W2, b2, Wfc, bfc):
    raise NotImplementedError("write your pallas kernel here")


# speedup vs baseline: 35.7139x; 35.7139x over previous
import functools

import jax
import jax.numpy as jnp
from jax import lax
from jax.experimental import pallas as pl
from jax.experimental.pallas import tpu as pltpu
from jax.experimental.pallas import tpu_sc as plsc

N = 10000
E = 320000
DF = 128
H = 16
G = 64
NCLS = 2

NC = 2
NS = 16
NW = NC * NS
LW = 128
EP = 327680
ROWS = EP // LW
RPW = ROWS // NW
CR = 8
CHUNKS = RPW // CR
NPAD = 10112
RPS = NPAD // NS
DPAD = 10240
DPS = DPAD // NS

_sc_mesh = plsc.VectorSubcoreMesh(
    core_axis_name="c", subcore_axis_name="s", num_cores=NC, num_subcores=NS)

_sc_params = pltpu.CompilerParams(use_tc_tiling_on_sc=False)


def _deg_body(dstr, out, didx, ones_v, zbuf, dacc):
    c = lax.axis_index("c")
    s = lax.axis_index("s")
    w = c * NS + s

    def _zero(j, carry):
        zbuf[pl.ds(j * 16, 16)] = jnp.zeros((16,), jnp.float32)
        return carry

    lax.fori_loop(0, DPS // 16, _zero, 0)

    def _one(j, carry):
        ones_v[pl.ds(j * 16, 16)] = jnp.ones((16,), jnp.float32)
        return carry

    lax.fori_loop(0, LW // 16, _one, 0)

    pltpu.sync_copy(zbuf, dacc.at[pl.ds(s * DPS, DPS)])
    plsc.subcore_barrier()

    def _chunk(ci, carry):
        base = w * RPW + ci * CR
        pltpu.sync_copy(dstr.at[pl.ds(base, CR)], didx)
        for j in range(CR):
            pltpu.sync_copy(ones_v, dacc.at[didx.at[j]], add=True)
        return carry

    lax.fori_loop(0, CHUNKS, _chunk, 0)
    plsc.subcore_barrier()
    pltpu.sync_copy(dacc.at[pl.ds(s * DPS, DPS)], out.at[c, pl.ds(s * DPS, DPS)])


_deg_kernel = pl.kernel(
    _deg_body,
    out_type=jax.ShapeDtypeStruct((NC, DPAD), jnp.float32),
    mesh=_sc_mesh,
    scratch_types=[
        pltpu.VMEM((CR, LW), jnp.int32),
        pltpu.VMEM((LW,), jnp.float32),
        pltpu.VMEM((DPS,), jnp.float32),
        pltpu.VMEM_SHARED((DPAD,), jnp.float32),
    ],
    compiler_params=_sc_params,
)


def _conv_body(g, srcr, dstr, out, sidx, didx, rows, zbuf, acc, sems):
    c = lax.axis_index("c")
    s = lax.axis_index("s")
    w = c * NS + s

    def _zero(j, carry):
        zbuf[j, :] = jnp.zeros((16,), jnp.float32)
        return carry

    lax.fori_loop(0, RPS, _zero, 0)
    pltpu.sync_copy(zbuf, acc.at[pl.ds(s * RPS, RPS)])
    plsc.subcore_barrier()

    def _chunk(ci, carry):
        base = w * RPW + ci * CR
        pltpu.sync_copy(srcr.at[pl.ds(base, CR)], sidx)
        pltpu.sync_copy(dstr.at[pl.ds(base, CR)], didx)
        copies = [
            pltpu.async_copy(g.at[sidx.at[j]],
                             rows.at[pl.ds(j * LW, LW)], sems.at[j])
            for j in range(CR)
        ]
        for j in range(CR):
            copies[j].wait()
            pltpu.sync_copy(rows.at[pl.ds(j * LW, LW)],
                            acc.at[didx.at[j]], add=True)
        return carry

    lax.fori_loop(0, CHUNKS, _chunk, 0)
    plsc.subcore_barrier()
    pltpu.sync_copy(acc.at[pl.ds(s * RPS, RPS)], out.at[c, pl.ds(s * RPS, RPS)])


_conv_kernel = pl.kernel(
    _conv_body,
    out_type=jax.ShapeDtypeStruct((NC, NPAD, H), jnp.float32),
    mesh=_sc_mesh,
    scratch_types=[
        pltpu.VMEM((CR, LW), jnp.int32),
        pltpu.VMEM((CR, LW), jnp.int32),
        pltpu.VMEM((CR * LW, H), jnp.float32),
        pltpu.VMEM((RPS, H), jnp.float32),
        pltpu.VMEM_SHARED((NPAD, H), jnp.float32),
        pltpu.SemaphoreType.DMA((CR,)),
    ],
    compiler_params=_sc_params,
)


def _tc1_body(x_ref, w1_ref, degt_ref, g_ref, dinv_ref):
    deg = degt_ref[:, 0:1] + degt_ref[:, 1:2] + 1.0
    dinv = lax.rsqrt(deg)
    dinv_ref[...] = dinv
    h = jnp.dot(x_ref[...], w1_ref[...], preferred_element_type=jnp.float32)
    g_ref[...] = h * dinv


_tc1 = pl.pallas_call(
    _tc1_body,
    out_shape=(
        jax.ShapeDtypeStruct((N, H), jnp.float32),
        jax.ShapeDtypeStruct((N, 1), jnp.float32),
    ),
)


def _tc2_body(s1_ref, g1_ref, dinv_ref, b1_ref, w2_ref, g2_ref):
    ssum = s1_ref[0, :N, :] + s1_ref[1, :N, :] + g1_ref[...]
    t = jnp.maximum(dinv_ref[...] * ssum + b1_ref[...], 0.0)
    g2_ref[...] = dinv_ref[...] * jnp.dot(
        t, w2_ref[...], preferred_element_type=jnp.float32)


_tc2 = pl.pallas_call(
    _tc2_body,
    out_shape=jax.ShapeDtypeStruct((N, H), jnp.float32),
)


def _tc3_body(s2_ref, g2_ref, dinv_ref, b2_ref, batch_ref, wfc_ref, bfc_ref,
              out_ref):
    ssum = s2_ref[0, :N, :] + s2_ref[1, :N, :] + g2_ref[...]
    o = jnp.maximum(dinv_ref[...] * ssum + b2_ref[...], 0.0)
    m = (batch_ref[...] == lax.broadcasted_iota(jnp.int32, (1, G), 1)
         ).astype(jnp.float32)
    sums = lax.dot_general(m, o, (((0,), (0,)), ((), ())),
                           preferred_element_type=jnp.float32)
    ones = jnp.ones((N, 1), jnp.float32)
    cnts = lax.dot_general(m, ones, (((0,), (0,)), ((), ())),
                           preferred_element_type=jnp.float32)
    pooled = sums / jnp.maximum(cnts, 1.0)
    out_ref[...] = jnp.dot(pooled, wfc_ref[...],
                           preferred_element_type=jnp.float32) + bfc_ref[...]


_tc3 = pl.pallas_call(
    _tc3_body,
    out_shape=jax.ShapeDtypeStruct((G, NCLS), jnp.float32),
)


@jax.jit
def kernel(x, edge_index, batch, W1, b1, W2, b2, Wfc, bfc):
    src = edge_index[0]
    dst = edge_index[1]
    srcr = jnp.concatenate(
        [src, jnp.zeros((EP - E,), jnp.int32)]).reshape(ROWS, LW)
    dstr = jnp.concatenate(
        [dst, jnp.full((EP - E,), N, jnp.int32)]).reshape(ROWS, LW)

    degp = _deg_kernel(dstr)
    degt = degp.T[:N]

    g1, dinv = _tc1(x, W1, degt)
    S1 = _conv_kernel(g1, srcr, dstr)
    g2 = _tc2(S1, g1, dinv, b1.reshape(1, H), W2)
    S2 = _conv_kernel(g2, srcr, dstr)
    out = _tc3(S2, g2, dinv, b2.reshape(1, H), batch.reshape(N, 1),
               Wfc, bfc.reshape(1, NCLS))
    return out
